# baseline (speedup 1.0000x reference)
import functools

import jax
import jax.numpy as jnp
from jax import lax
from jax.experimental import pallas as pl
from jax.experimental.pallas import tpu as pltpu
from jax.experimental.pallas import tpu_sc as plsc

D1 = 200
D2 = 128
B = 1048576
NC = 2
NS = 16
NW = NC * NS
BPW = B // NW
CHUNK = 8192
NCHUNK = BPW // CHUNK
VEC = 16


def _flat_gather(x_flat, i0, i1, i2):
    mesh = plsc.VectorSubcoreMesh(core_axis_name="c", subcore_axis_name="s")

    @functools.partial(
        pl.kernel,
        out_type=jax.ShapeDtypeStruct((B,), jnp.float32),
        mesh=mesh,
        scratch_types=[
            pltpu.VMEM((CHUNK,), jnp.int32),
            pltpu.VMEM((CHUNK,), jnp.int32),
            pltpu.VMEM((CHUNK,), jnp.int32),
            pltpu.VMEM((CHUNK,), jnp.int32),
            pltpu.VMEM((CHUNK,), jnp.float32),
            pltpu.SemaphoreType.DMA,
        ],
    )
    def k(x_hbm, i0_hbm, i1_hbm, i2_hbm, out_hbm,
          i0_v, i1_v, i2_v, flat_v, vals_v, sem):
        wid = lax.axis_index("s") * NC + lax.axis_index("c")
        base = wid * BPW

        def chunk_body(c, carry):
            off = base + c * CHUNK
            pltpu.sync_copy(i0_hbm.at[pl.ds(off, CHUNK)], i0_v)
            pltpu.sync_copy(i1_hbm.at[pl.ds(off, CHUNK)], i1_v)
            pltpu.sync_copy(i2_hbm.at[pl.ds(off, CHUNK)], i2_v)

            def vec_body(i, acc):
                s = pl.ds(i * VEC, VEC)
                flat_v[s] = i0_v[s] * (D1 * D2) + i1_v[s] * D2 + i2_v[s]
                return acc

            lax.fori_loop(0, CHUNK // VEC, vec_body, 0, unroll=8)

            pltpu.async_copy(x_hbm.at[flat_v], vals_v, sem).wait()
            pltpu.sync_copy(vals_v, out_hbm.at[pl.ds(off, CHUNK)])
            return carry

        lax.fori_loop(0, NCHUNK, chunk_body, 0)

    return k(x_flat, i0, i1, i2)


@jax.jit
def kernel(x, idx0, idx1, idx2):
    x_flat = x.reshape(-1)
    return _flat_gather(
        x_flat,
        idx0.astype(jnp.int32),
        idx1.astype(jnp.int32),
        idx2.astype(jnp.int32),
    )


# speedup vs baseline: 1.3245x; 1.3245x over previous
import functools

import jax
import jax.numpy as jnp
from jax import lax
from jax.experimental import pallas as pl
from jax.experimental.pallas import tpu as pltpu
from jax.experimental.pallas import tpu_sc as plsc

D1 = 200
D2 = 128
B = 1048576
NC = 2
NS = 16
NW = NC * NS
BPW = B // NW
CHUNK = 8192
NCHUNK = BPW // CHUNK
VEC = 16


def _flat_gather(x_flat, i0, i1, i2):
    mesh = plsc.VectorSubcoreMesh(core_axis_name="c", subcore_axis_name="s")

    @functools.partial(
        pl.kernel,
        out_type=jax.ShapeDtypeStruct((B,), jnp.float32),
        mesh=mesh,
        scratch_types=[
            pltpu.VMEM((CHUNK,), jnp.int32),
            pltpu.VMEM((CHUNK,), jnp.int32),
            pltpu.VMEM((CHUNK,), jnp.int32),
            pltpu.VMEM((CHUNK,), jnp.int32),
            pltpu.VMEM((CHUNK,), jnp.int32),
            pltpu.VMEM((CHUNK,), jnp.int32),
            pltpu.VMEM((BPW,), jnp.int32),
            pltpu.VMEM((BPW,), jnp.float32),
            pltpu.SemaphoreType.DMA,
            pltpu.SemaphoreType.DMA,
        ],
    )
    def k(x_hbm, i0_hbm, i1_hbm, i2_hbm, out_hbm,
          i0a, i1a, i2a, i0b, i1b, i2b, flat_v, vals_v, sem_i, sem_g):
        wid = lax.axis_index("s") * NC + lax.axis_index("c")
        base = wid * BPW
        bufs = ((i0a, i1a, i2a), (i0b, i1b, i2b))
        idx_hbm = (i0_hbm, i1_hbm, i2_hbm)

        descs = [
            pltpu.async_copy(h.at[pl.ds(base, CHUNK)], r, sem_i)
            for h, r in zip(idx_hbm, bufs[0])
        ]
        gathers = []
        for c in range(NCHUNK):
            cur = bufs[c % 2]
            for d in descs:
                d.wait()
            if c + 1 < NCHUNK:
                off_next = base + (c + 1) * CHUNK
                descs = [
                    pltpu.async_copy(h.at[pl.ds(off_next, CHUNK)], r, sem_i)
                    for h, r in zip(idx_hbm, bufs[(c + 1) % 2])
                ]
            fo = c * CHUNK
            c0, c1, c2 = cur

            @plsc.parallel_loop(0, CHUNK, VEC, unroll=8)
            def _(i):
                s = pl.ds(i, VEC)
                flat_v[pl.ds(fo + i, VEC)] = c0[s] * (D1 * D2) + c1[s] * D2 + c2[s]

            gathers.append(
                pltpu.async_copy(
                    x_hbm.at[flat_v.at[pl.ds(fo, CHUNK)]],
                    vals_v.at[pl.ds(fo, CHUNK)],
                    sem_g,
                )
            )
        for g in gathers:
            g.wait()
        pltpu.sync_copy(vals_v, out_hbm.at[pl.ds(base, BPW)])

    return k(x_flat, i0, i1, i2)


@jax.jit
def kernel(x, idx0, idx1, idx2):
    x_flat = x.reshape(-1)
    return _flat_gather(
        x_flat,
        idx0.astype(jnp.int32),
        idx1.astype(jnp.int32),
        idx2.astype(jnp.int32),
    )


# speedup vs baseline: 1.3446x; 1.0152x over previous
import functools

import jax
import jax.numpy as jnp
from jax import lax
from jax.experimental import pallas as pl
from jax.experimental.pallas import tpu as pltpu
from jax.experimental.pallas import tpu_sc as plsc

D1 = 200
D2 = 128
B = 1048576
NC = 2
NS = 16
NW = NC * NS
BPW = B // NW
VEC = 16

SCHED = (2048, 6144, 8192, 8192, 8192)
NCH = len(SCHED)
OFF = tuple(sum(SCHED[:c]) for c in range(NCH))
STAGE = max(SCHED)


def _flat_gather(x_flat, i0, i1, i2):
    mesh = plsc.VectorSubcoreMesh(core_axis_name="c", subcore_axis_name="s")

    @functools.partial(
        pl.kernel,
        out_type=jax.ShapeDtypeStruct((B,), jnp.float32),
        mesh=mesh,
        scratch_types=[
            pltpu.VMEM((STAGE,), jnp.int32),
            pltpu.VMEM((STAGE,), jnp.int32),
            pltpu.VMEM((STAGE,), jnp.int32),
            pltpu.VMEM((STAGE,), jnp.int32),
            pltpu.VMEM((BPW,), jnp.int32),
            pltpu.VMEM((BPW,), jnp.float32),
            pltpu.SemaphoreType.DMA,
            pltpu.SemaphoreType.DMA,
            [pltpu.SemaphoreType.DMA] * NCH,
            pltpu.SemaphoreType.DMA,
        ],
    )
    def k(x_hbm, i0_hbm, i1_hbm, i2_hbm, out_hbm,
          i0a, i1a, i0b, i1b, flat_v, vals_v, sem_a, sem_b, gsems, sem_w):
        wid = lax.axis_index("s") * NC + lax.axis_index("c")
        base = wid * BPW
        slot_bufs = ((i0a, i1a), (i0b, i1b))
        slot_sems = (sem_a, sem_b)

        def fire_loads(c):
            off = base + OFF[c]
            n = SCHED[c]
            b0, b1 = slot_bufs[c % 2]
            sem = slot_sems[c % 2]
            return [
                pltpu.async_copy(i0_hbm.at[pl.ds(off, n)], b0.at[pl.ds(0, n)], sem),
                pltpu.async_copy(i1_hbm.at[pl.ds(off, n)], b1.at[pl.ds(0, n)], sem),
                pltpu.async_copy(i2_hbm.at[pl.ds(off, n)],
                                 flat_v.at[pl.ds(OFF[c], n)], sem),
            ]

        pending = {0: fire_loads(0), 1: fire_loads(1)}
        gathers = []
        for c in range(NCH):
            for d in pending.pop(c):
                d.wait()
            b0, b1 = slot_bufs[c % 2]
            fo = OFF[c]

            @plsc.parallel_loop(0, SCHED[c], VEC, unroll=8)
            def _(i):
                s = pl.ds(i, VEC)
                f = pl.ds(fo + i, VEC)
                flat_v[f] = b0[s] * (D1 * D2) + b1[s] * D2 + flat_v[f]

            gathers.append(
                pltpu.async_copy(
                    x_hbm.at[flat_v.at[pl.ds(fo, SCHED[c])]],
                    vals_v.at[pl.ds(fo, SCHED[c])],
                    gsems[c],
                )
            )
            if c + 2 < NCH:
                pending[c + 2] = fire_loads(c + 2)

        writebacks = []
        for c in range(NCH):
            gathers[c].wait()
            writebacks.append(
                pltpu.async_copy(
                    vals_v.at[pl.ds(OFF[c], SCHED[c])],
                    out_hbm.at[pl.ds(base + OFF[c], SCHED[c])],
                    sem_w,
                )
            )
        for d in writebacks:
            d.wait()

    return k(x_flat, i0, i1, i2)


@jax.jit
def kernel(x, idx0, idx1, idx2):
    x_flat = x.reshape(-1)
    return _flat_gather(
        x_flat,
        idx0.astype(jnp.int32),
        idx1.astype(jnp.int32),
        idx2.astype(jnp.int32),
    )
